# baseline (speedup 1.0000x reference)
import jax
import jax.numpy as jnp
from jax import lax
from jax.experimental import pallas as pl
from jax.experimental.pallas import tpu as pltpu
from jax.experimental.pallas import tpu_sc as plsc

N = 10000
D = 128
E = 320000
NC, NS, L = 2, 16, 16
NW = NC * NS
DW = 8
CL = 40
NCH = 250
RPT = N // NS
ZR = 64


def _sc_body(x_hbm, edges4, ones_hbm, zero_hbm, parts, degp,
             idx_r, idx_c, buf0, buf1, ones_v,
             acc, deg, semg0, semg1, semd):
    c = lax.axis_index("c")
    s = lax.axis_index("s")
    wid = c * NS + s

    pltpu.sync_copy(edges4.at[0, wid], idx_r)
    pltpu.sync_copy(edges4.at[1, wid], idx_c)
    pltpu.sync_copy(ones_hbm, ones_v)

    base = s * RPT
    for k in range(9):
        pltpu.sync_copy(zero_hbm, acc.at[pl.ds(base + k * ZR, ZR)])
        pltpu.sync_copy(zero_hbm.at[:, :DW], deg.at[pl.ds(base + k * ZR, ZR)])
    pltpu.sync_copy(zero_hbm.at[pl.ds(0, 49)], acc.at[pl.ds(base + 9 * ZR, 49)])
    pltpu.sync_copy(zero_hbm.at[pl.ds(0, 49), :DW],
                    deg.at[pl.ds(base + 9 * ZR, 49)])
    plsc.subcore_barrier()

    pltpu.async_copy(x_hbm.at[idx_c.at[0]], buf0, semg0)
    pltpu.async_copy(x_hbm.at[idx_c.at[1]], buf1, semg1)

    def step(j, buf, semg):
        pltpu.make_async_copy(x_hbm.at[idx_c.at[j]], buf, semg).wait()
        dsc = pltpu.async_copy(ones_v, deg.at[idx_r.at[j]], semd, add=True)
        pltpu.sync_copy(buf, acc.at[idx_r.at[j]], add=True)
        dsc.wait()

    def body(i, carry):
        j0 = 2 * i
        step(j0, buf0, semg0)
        pltpu.async_copy(x_hbm.at[idx_c.at[j0 + 2]], buf0, semg0)
        step(j0 + 1, buf1, semg1)
        pltpu.async_copy(x_hbm.at[idx_c.at[j0 + 3]], buf1, semg1)
        return carry

    lax.fori_loop(0, (NCH - 3) // 2, body, 0)

    step(NCH - 3, buf0, semg0)
    pltpu.async_copy(x_hbm.at[idx_c.at[NCH - 1]], buf0, semg0)
    step(NCH - 2, buf1, semg1)
    step(NCH - 1, buf0, semg0)

    plsc.subcore_barrier()
    pltpu.sync_copy(acc.at[pl.ds(base, RPT)], parts.at[c, pl.ds(base, RPT)])
    pltpu.sync_copy(deg.at[pl.ds(base, RPT)], degp.at[c, pl.ds(base, RPT)])


_sc_scatter = pl.kernel(
    _sc_body,
    out_type=[
        jax.ShapeDtypeStruct((NC, N, D), jnp.float32),
        jax.ShapeDtypeStruct((NC, N, DW), jnp.float32),
    ],
    mesh=plsc.VectorSubcoreMesh(core_axis_name="c", subcore_axis_name="s",
                                num_cores=NC, num_subcores=NS),
    scratch_types=[
        pltpu.VMEM((NCH, CL), jnp.int32),
        pltpu.VMEM((NCH, CL), jnp.int32),
        pltpu.VMEM((CL, D), jnp.float32),
        pltpu.VMEM((CL, D), jnp.float32),
        pltpu.VMEM((CL, DW), jnp.float32),
        pltpu.VMEM_SHARED((N, D), jnp.float32),
        pltpu.VMEM_SHARED((N, DW), jnp.float32),
        pltpu.SemaphoreType.DMA,
        pltpu.SemaphoreType.DMA,
        pltpu.SemaphoreType.DMA,
    ],
    compiler_params=pltpu.CompilerParams(use_tc_tiling_on_sc=False),
)


def _combine_body(rate_ref, x_ref, p_ref, d_ref, o_ref):
    x = x_ref[...]
    ssum = p_ref[0] + p_ref[1]
    deg = d_ref[0, :, :1] + d_ref[1, :, :1]
    deg = jnp.maximum(deg, 1.0)
    r = jnp.clip(rate_ref[0], 0.0, 0.2)
    o_ref[...] = jnp.maximum(x - r * (ssum / deg), 0.0)


_BR = 1000


def _combine(x, parts, degp, rate):
    return pl.pallas_call(
        _combine_body,
        grid=(N // _BR,),
        in_specs=[
            pl.BlockSpec(memory_space=pltpu.SMEM),
            pl.BlockSpec((_BR, D), lambda i: (i, 0)),
            pl.BlockSpec((NC, _BR, D), lambda i: (0, i, 0)),
            pl.BlockSpec((NC, _BR, DW), lambda i: (0, i, 0)),
        ],
        out_specs=pl.BlockSpec((_BR, D), lambda i: (i, 0)),
        out_shape=jax.ShapeDtypeStruct((N, D), jnp.float32),
    )(rate, x, parts, degp)


@jax.jit
def kernel(x, edge_index, rate):
    edges4 = edge_index.reshape(2, NW, NCH, CL)
    ones_hbm = jnp.tile(jnp.eye(1, DW, dtype=jnp.float32), (CL, 1))
    zero_hbm = jnp.zeros((ZR, D), jnp.float32)

    parts, degp = _sc_scatter(x, edges4, ones_hbm, zero_hbm)
    return _combine(x, parts, degp, rate)


# speedup vs baseline: 1.2606x; 1.2606x over previous
import jax
import jax.numpy as jnp
from jax import lax
from jax.experimental import pallas as pl
from jax.experimental.pallas import tpu as pltpu
from jax.experimental.pallas import tpu_sc as plsc

N = 10000
D = 128
E = 320000
NC, NS, L = 2, 16, 16
NW = NC * NS
EPT = E // NW
DW = 8
CL = 96
NF = EPT // CL
TL = EPT - NF * CL
RPT = N // NS
ZR = 64


def _sc_body(x_hbm, e_hbm, ones_hbm, zero_hbm, parts, degp,
             idx_r, idx_c, buf0, buf1, ones_v, acc, deg,
             semg0, semg1, semd):
    c = lax.axis_index("c")
    s = lax.axis_index("s")
    wid = c * NS + s

    pltpu.sync_copy(e_hbm.at[0, pl.ds(wid * EPT, EPT)], idx_r)
    pltpu.sync_copy(e_hbm.at[1, pl.ds(wid * EPT, EPT)], idx_c)
    pltpu.sync_copy(ones_hbm, ones_v)

    base = s * RPT
    for k in range(9):
        pltpu.sync_copy(zero_hbm, acc.at[pl.ds(base + k * ZR, ZR)])
        pltpu.sync_copy(zero_hbm.at[:, :DW], deg.at[pl.ds(base + k * ZR, ZR)])
    pltpu.sync_copy(zero_hbm.at[pl.ds(0, 49)], acc.at[pl.ds(base + 9 * ZR, 49)])
    pltpu.sync_copy(zero_hbm.at[pl.ds(0, 49), :DW],
                    deg.at[pl.ds(base + 9 * ZR, 49)])
    plsc.subcore_barrier()

    def cidx(j):
        return idx_c.at[pl.ds(j * CL, CL)]

    def ridx(j):
        return idx_r.at[pl.ds(j * CL, CL)]

    pltpu.async_copy(x_hbm.at[cidx(0)], buf0, semg0)
    pltpu.async_copy(x_hbm.at[cidx(1)], buf1, semg1)

    def step(j, buf, semg):
        pltpu.make_async_copy(x_hbm.at[cidx(j)], buf, semg).wait()
        dsc = pltpu.async_copy(ones_v, deg.at[ridx(j)], semd, add=True)
        pltpu.sync_copy(buf, acc.at[ridx(j)], add=True)
        dsc.wait()

    def body(i, carry):
        j0 = 2 * i
        step(j0, buf0, semg0)
        pltpu.async_copy(x_hbm.at[cidx(j0 + 2)], buf0, semg0)
        step(j0 + 1, buf1, semg1)
        pltpu.async_copy(x_hbm.at[cidx(j0 + 3)], buf1, semg1)
        return carry

    lax.fori_loop(0, (NF - 2) // 2, body, 0)

    step(NF - 2, buf0, semg0)
    pltpu.async_copy(x_hbm.at[idx_c.at[pl.ds(NF * CL, TL)]],
                     buf0.at[pl.ds(0, TL)], semg0)
    step(NF - 1, buf1, semg1)
    pltpu.make_async_copy(x_hbm.at[idx_c.at[pl.ds(NF * CL, TL)]],
                          buf0.at[pl.ds(0, TL)], semg0).wait()
    dsc = pltpu.async_copy(ones_v.at[pl.ds(0, TL)],
                           deg.at[idx_r.at[pl.ds(NF * CL, TL)]],
                           semd, add=True)
    pltpu.sync_copy(buf0.at[pl.ds(0, TL)],
                    acc.at[idx_r.at[pl.ds(NF * CL, TL)]], add=True)
    dsc.wait()

    plsc.subcore_barrier()
    pltpu.sync_copy(acc.at[pl.ds(base, RPT)], parts.at[c, pl.ds(base, RPT)])
    pltpu.sync_copy(deg.at[pl.ds(base, RPT)], degp.at[c, pl.ds(base, RPT)])


_sc_scatter = pl.kernel(
    _sc_body,
    out_type=[
        jax.ShapeDtypeStruct((NC, N, D), jnp.float32),
        jax.ShapeDtypeStruct((NC, N, DW), jnp.float32),
    ],
    mesh=plsc.VectorSubcoreMesh(core_axis_name="c", subcore_axis_name="s",
                                num_cores=NC, num_subcores=NS),
    scratch_types=[
        pltpu.VMEM((EPT,), jnp.int32),
        pltpu.VMEM((EPT,), jnp.int32),
        pltpu.VMEM((CL, D), jnp.float32),
        pltpu.VMEM((CL, D), jnp.float32),
        pltpu.VMEM((CL, DW), jnp.float32),
        pltpu.VMEM_SHARED((N, D), jnp.float32),
        pltpu.VMEM_SHARED((N, DW), jnp.float32),
        pltpu.SemaphoreType.DMA,
        pltpu.SemaphoreType.DMA,
        pltpu.SemaphoreType.DMA,
    ],
    compiler_params=pltpu.CompilerParams(use_tc_tiling_on_sc=False),
)


def _combine_body(rate_ref, x_ref, p_ref, d_ref, o_ref):
    x = x_ref[...]
    ssum = p_ref[0] + p_ref[1]
    deg = d_ref[0, :, :1] + d_ref[1, :, :1]
    deg = jnp.maximum(deg, 1.0)
    r = jnp.clip(rate_ref[0], 0.0, 0.2)
    o_ref[...] = jnp.maximum(x - r * (ssum / deg), 0.0)


_BR = 1000


def _combine(x, parts, degp, rate):
    return pl.pallas_call(
        _combine_body,
        grid=(N // _BR,),
        in_specs=[
            pl.BlockSpec(memory_space=pltpu.SMEM),
            pl.BlockSpec((_BR, D), lambda i: (i, 0)),
            pl.BlockSpec((NC, _BR, D), lambda i: (0, i, 0)),
            pl.BlockSpec((NC, _BR, DW), lambda i: (0, i, 0)),
        ],
        out_specs=pl.BlockSpec((_BR, D), lambda i: (i, 0)),
        out_shape=jax.ShapeDtypeStruct((N, D), jnp.float32),
    )(rate, x, parts, degp)


@jax.jit
def kernel(x, edge_index, rate):
    ones_hbm = jnp.tile(jnp.eye(1, DW, dtype=jnp.float32), (CL, 1))
    zero_hbm = jnp.zeros((ZR, D), jnp.float32)

    parts, degp = _sc_scatter(x, edge_index, ones_hbm, zero_hbm)
    return _combine(x, parts, degp, rate)


# speedup vs baseline: 1.3705x; 1.0872x over previous
import jax
import jax.numpy as jnp
from jax import lax
from jax.experimental import pallas as pl
from jax.experimental.pallas import tpu as pltpu
from jax.experimental.pallas import tpu_sc as plsc

N = 10000
D = 128
E = 320000
NC, NS, L = 2, 16, 16
NW = NC * NS
EPT = E // NW
DW = 8
CL = 128
NF = EPT // CL
TL = EPT - NF * CL
RPT = N // NS
ZR = 64


def _sc_body(x_hbm, e_hbm, ones_hbm, zacc_hbm, zdeg_hbm, parts, degp,
             idx_r, idx_c, buf0, buf1, ones_v, acc, deg,
             semg0, semg1, semd):
    c = lax.axis_index("c")
    s = lax.axis_index("s")
    wid = c * NS + s

    pltpu.sync_copy(e_hbm.at[0, pl.ds(wid * EPT, EPT)], idx_r)
    pltpu.sync_copy(e_hbm.at[1, pl.ds(wid * EPT, EPT)], idx_c)
    pltpu.sync_copy(ones_hbm, ones_v)

    base = s * RPT
    for k in range(9):
        pltpu.sync_copy(zacc_hbm, acc.at[pl.ds(base + k * ZR, ZR)])
        pltpu.sync_copy(zdeg_hbm, deg.at[pl.ds(base + k * ZR, ZR)])
    pltpu.sync_copy(zacc_hbm.at[pl.ds(0, 49)], acc.at[pl.ds(base + 9 * ZR, 49)])
    pltpu.sync_copy(zdeg_hbm.at[pl.ds(0, 49)],
                    deg.at[pl.ds(base + 9 * ZR, 49)])
    plsc.subcore_barrier()

    def cidx(j):
        return idx_c.at[pl.ds(j * CL, CL)]

    def ridx(j):
        return idx_r.at[pl.ds(j * CL, CL)]

    pltpu.async_copy(x_hbm.at[cidx(0)], buf0, semg0)
    pltpu.async_copy(x_hbm.at[cidx(1)], buf1, semg1)

    def step(j, buf, semg):
        pltpu.make_async_copy(x_hbm.at[cidx(j)], buf, semg).wait()
        dsc = pltpu.async_copy(ones_v, deg.at[ridx(j)], semd, add=True)
        pltpu.sync_copy(buf, acc.at[ridx(j)], add=True)
        dsc.wait()

    def body(i, carry):
        j0 = 2 * i
        step(j0, buf0, semg0)
        pltpu.async_copy(x_hbm.at[cidx(j0 + 2)], buf0, semg0)
        step(j0 + 1, buf1, semg1)
        pltpu.async_copy(x_hbm.at[cidx(j0 + 3)], buf1, semg1)
        return carry

    lax.fori_loop(0, (NF - 2) // 2, body, 0)

    step(NF - 2, buf0, semg0)
    pltpu.async_copy(x_hbm.at[idx_c.at[pl.ds(NF * CL, TL)]],
                     buf0.at[pl.ds(0, TL)], semg0)
    step(NF - 1, buf1, semg1)
    pltpu.make_async_copy(x_hbm.at[idx_c.at[pl.ds(NF * CL, TL)]],
                          buf0.at[pl.ds(0, TL)], semg0).wait()
    dsc = pltpu.async_copy(ones_v.at[pl.ds(0, TL)],
                           deg.at[idx_r.at[pl.ds(NF * CL, TL)]],
                           semd, add=True)
    pltpu.sync_copy(buf0.at[pl.ds(0, TL)],
                    acc.at[idx_r.at[pl.ds(NF * CL, TL)]], add=True)
    dsc.wait()

    plsc.subcore_barrier()
    pltpu.sync_copy(acc.at[pl.ds(base, RPT)], parts.at[c, pl.ds(base, RPT)])
    pltpu.sync_copy(deg.at[pl.ds(base, RPT)], degp.at[c, pl.ds(base, RPT)])


_sc_scatter = pl.kernel(
    _sc_body,
    out_type=[
        jax.ShapeDtypeStruct((NC, N, D), jnp.bfloat16),
        jax.ShapeDtypeStruct((NC, N, DW), jnp.float32),
    ],
    mesh=plsc.VectorSubcoreMesh(core_axis_name="c", subcore_axis_name="s",
                                num_cores=NC, num_subcores=NS),
    scratch_types=[
        pltpu.VMEM((EPT,), jnp.int32),
        pltpu.VMEM((EPT,), jnp.int32),
        pltpu.VMEM((CL, D), jnp.bfloat16),
        pltpu.VMEM((CL, D), jnp.bfloat16),
        pltpu.VMEM((CL, DW), jnp.float32),
        pltpu.VMEM_SHARED((N, D), jnp.bfloat16),
        pltpu.VMEM_SHARED((N, DW), jnp.float32),
        pltpu.SemaphoreType.DMA,
        pltpu.SemaphoreType.DMA,
        pltpu.SemaphoreType.DMA,
    ],
    compiler_params=pltpu.CompilerParams(use_tc_tiling_on_sc=False),
)


def _combine_body(rate_ref, x_ref, p_ref, d_ref, o_ref):
    x = x_ref[...]
    ssum = p_ref[0].astype(jnp.float32) + p_ref[1].astype(jnp.float32)
    deg = d_ref[0, :, :1] + d_ref[1, :, :1]
    deg = jnp.maximum(deg, 1.0)
    r = jnp.clip(rate_ref[0], 0.0, 0.2)
    o_ref[...] = jnp.maximum(x - r * (ssum / deg), 0.0)


_BR = 1000


def _combine(x, parts, degp, rate):
    return pl.pallas_call(
        _combine_body,
        grid=(N // _BR,),
        in_specs=[
            pl.BlockSpec(memory_space=pltpu.SMEM),
            pl.BlockSpec((_BR, D), lambda i: (i, 0)),
            pl.BlockSpec((NC, _BR, D), lambda i: (0, i, 0)),
            pl.BlockSpec((NC, _BR, DW), lambda i: (0, i, 0)),
        ],
        out_specs=pl.BlockSpec((_BR, D), lambda i: (i, 0)),
        out_shape=jax.ShapeDtypeStruct((N, D), jnp.float32),
    )(rate, x, parts, degp)


@jax.jit
def kernel(x, edge_index, rate):
    xb = x.astype(jnp.bfloat16)
    ones_hbm = jnp.tile(jnp.eye(1, DW, dtype=jnp.float32), (CL, 1))
    zacc_hbm = jnp.zeros((ZR, D), jnp.bfloat16)
    zdeg_hbm = jnp.zeros((ZR, DW), jnp.float32)

    parts, degp = _sc_scatter(xb, edge_index, ones_hbm, zacc_hbm, zdeg_hbm)
    return _combine(x, parts, degp, rate)
